# baseline (speedup 1.0000x reference)
import jax
import jax.numpy as jnp
from jax.experimental import pallas as pl
from jax.experimental.pallas import tpu as pltpu

_EP = 128


def _moe_body(r_ref, x_ref, wr_ref, br_ref, we_ref, be_ref, out_ref,
              web_ref, stage_ref, sems):
    n_exp = web_ref.shape[0]
    first = pl.program_id(0) == 0

    @pl.when(first)
    def _():
        pltpu.make_async_copy(we_ref.at[0], stage_ref.at[0], sems.at[0]).start()

    rb = r_ref[...].astype(jnp.bfloat16)
    logits = jnp.dot(rb, wr_ref[...], preferred_element_type=jnp.float32)
    logits = logits + br_ref[...]
    m = jnp.max(logits, axis=-1, keepdims=True)
    u = jnp.exp(logits - m)
    denom = jnp.sum(u, axis=-1, keepdims=True)

    x = x_ref[...].astype(jnp.bfloat16)
    acc = jnp.dot(u.astype(jnp.bfloat16), be_ref[...],
                  preferred_element_type=jnp.float32)
    for e in range(n_exp):
        @pl.when(first)
        def _():
            if e + 1 < n_exp:
                pltpu.make_async_copy(we_ref.at[e + 1], stage_ref.at[(e + 1) % 2],
                                      sems.at[(e + 1) % 2]).start()
            pltpu.make_async_copy(we_ref.at[e], stage_ref.at[e % 2],
                                  sems.at[e % 2]).wait()
            web_ref[e] = stage_ref[e % 2].astype(jnp.bfloat16)

        acc = acc + u[:, e : e + 1] * jnp.dot(
            x, web_ref[e], preferred_element_type=jnp.float32)
    out_ref[...] = acc / denom


def kernel(router_input, x, Wr, br, We, be):
    n, d = x.shape
    n_exp = We.shape[0]
    bn = 512

    wrp = jnp.zeros((d, _EP), jnp.bfloat16).at[:, :n_exp].set(
        Wr.astype(jnp.bfloat16))
    brp = jnp.full((1, _EP), -jnp.inf, jnp.float32).at[0, :n_exp].set(br)
    bep = jnp.zeros((_EP, d), jnp.bfloat16).at[:n_exp].set(
        be.astype(jnp.bfloat16))

    return pl.pallas_call(
        _moe_body,
        grid=(n // bn,),
        in_specs=[
            pl.BlockSpec((bn, d), lambda i: (i, 0)),
            pl.BlockSpec((bn, d), lambda i: (i, 0)),
            pl.BlockSpec((d, _EP), lambda i: (0, 0)),
            pl.BlockSpec((1, _EP), lambda i: (0, 0)),
            pl.BlockSpec(memory_space=pl.ANY),
            pl.BlockSpec((_EP, d), lambda i: (0, 0)),
        ],
        out_specs=pl.BlockSpec((bn, d), lambda i: (i, 0)),
        out_shape=jax.ShapeDtypeStruct((n, d), jnp.float32),
        scratch_shapes=[
            pltpu.VMEM((n_exp, d, d), jnp.bfloat16),
            pltpu.VMEM((2, d, d), jnp.float32),
            pltpu.SemaphoreType.DMA((2,)),
        ],
        compiler_params=pltpu.CompilerParams(
            dimension_semantics=("arbitrary",),
            vmem_limit_bytes=100 * 1024 * 1024,
        ),
    )(router_input, x, wrp, brp, We, bep)


# speedup vs baseline: 1.1232x; 1.1232x over previous
import jax
import jax.numpy as jnp
from jax.experimental import pallas as pl
from jax.experimental.pallas import tpu as pltpu

_EP = 128


def _moe_body(r_ref, x_ref, wr_ref, br_ref, we_ref, be_ref, out_ref):
    n_exp = we_ref.shape[0]
    rb = r_ref[...].astype(jnp.bfloat16)
    logits = jnp.dot(rb, wr_ref[...], preferred_element_type=jnp.float32)
    logits = logits + br_ref[...]
    m = jnp.max(logits, axis=-1, keepdims=True)
    u = jnp.exp(logits - m)
    denom = jnp.sum(u, axis=-1, keepdims=True)

    x = x_ref[...].astype(jnp.bfloat16)
    acc = jnp.dot(u.astype(jnp.bfloat16), be_ref[...],
                  preferred_element_type=jnp.float32)
    for e in range(n_exp):
        w = we_ref[e].astype(jnp.bfloat16)
        acc = acc + u[:, e : e + 1] * jnp.dot(
            x, w, preferred_element_type=jnp.float32)
    out_ref[...] = acc / denom


def kernel(router_input, x, Wr, br, We, be):
    n, d = x.shape
    n_exp = We.shape[0]
    bn = 512

    wrp = jnp.zeros((d, _EP), jnp.bfloat16).at[:, :n_exp].set(
        Wr.astype(jnp.bfloat16))
    brp = jnp.full((1, _EP), -jnp.inf, jnp.float32).at[0, :n_exp].set(br)
    bep = jnp.zeros((_EP, d), jnp.bfloat16).at[:n_exp].set(
        be.astype(jnp.bfloat16))

    return pl.pallas_call(
        _moe_body,
        grid=(n // bn,),
        in_specs=[
            pl.BlockSpec((bn, d), lambda i: (i, 0)),
            pl.BlockSpec((bn, d), lambda i: (i, 0)),
            pl.BlockSpec((d, _EP), lambda i: (0, 0)),
            pl.BlockSpec((1, _EP), lambda i: (0, 0)),
            pl.BlockSpec((n_exp, d, d), lambda i: (0, 0, 0)),
            pl.BlockSpec((_EP, d), lambda i: (0, 0)),
        ],
        out_specs=pl.BlockSpec((bn, d), lambda i: (i, 0)),
        out_shape=jax.ShapeDtypeStruct((n, d), jnp.float32),
        compiler_params=pltpu.CompilerParams(
            dimension_semantics=("arbitrary",),
        ),
    )(router_input, x, wrp, brp, We, bep)
